# baseline (speedup 1.0000x reference)
import jax
import jax.numpy as jnp
from jax.experimental import pallas as pl
from jax.experimental.pallas import tpu as pltpu

_ALPHA = 10.0
_K = 10
_BIG = 3.0e38


def _shift_cols(a, s, n):
    if s == 0:
        return a
    c = a.shape[0]
    if s > 0:
        return jnp.concatenate([a[:, s:], jnp.zeros((c, s), a.dtype)], axis=1)
    return jnp.concatenate([jnp.zeros((c, -s), a.dtype), a[:, :s]], axis=1)


def _enc_body(x1_ref, w1_ref, b1_ref, g1_ref, be1_ref,
              w29_ref, b2_ref, g2_ref, be2_ref, w3_ref, b3_ref, o_ref):
    n = x1_ref.shape[1]
    h = jnp.dot(w1_ref[...], x1_ref[...],
                preferred_element_type=jnp.float32)
    h = h + b1_ref[...]
    m = jnp.mean(h, axis=1, keepdims=True)
    v = jnp.mean((h - m) ** 2, axis=1, keepdims=True)
    h = (h - m) / jnp.sqrt(v + 1e-5) * g1_ref[...] + be1_ref[...]
    h = jnp.maximum(h, 0.0)

    col = jax.lax.broadcasted_iota(jnp.int32, (1, n), 1)
    xc = col % 32
    yc = (col // 32) % 32

    acc = jnp.zeros((32, n), jnp.float32)
    j = 0
    for ey in (-1, 0, 1):
        my = ((yc + ey) >= 0) & ((yc + ey) < 32)
        for ex in (-1, 0, 1):
            mask = (my & ((xc + ex) >= 0) & ((xc + ex) < 32)).astype(h.dtype)
            sxy = _shift_cols(h, 32 * ey + ex, n) * mask
            stk = jnp.concatenate(
                [_shift_cols(sxy, 1024 * ez, n) for ez in (-1, 0, 1)], axis=0)
            acc = acc + jnp.dot(w29_ref[32 * j:32 * (j + 1), :], stk,
                                preferred_element_type=jnp.float32)
            j += 1

    h2 = acc + b2_ref[...]
    m = jnp.mean(h2, axis=1, keepdims=True)
    v = jnp.mean((h2 - m) ** 2, axis=1, keepdims=True)
    h2 = (h2 - m) / jnp.sqrt(v + 1e-5) * g2_ref[...] + be2_ref[...]
    h2 = jnp.maximum(h2, 0.0)

    lat = jnp.dot(w3_ref[...], h2,
                  preferred_element_type=jnp.float32)
    lat = lat + b3_ref[...]
    norm = jnp.sqrt(jnp.sum(lat * lat, axis=0, keepdims=True))
    o_ref[...] = lat / jnp.maximum(norm, 1e-12)


def _knn_body(q_ref, k_ref, lo_ref, o_ref):
    q = q_ref[...]
    keys = k_ref[...]
    qn = jnp.sum(q * q, axis=0, keepdims=True)
    kn = jnp.sum(keys * keys, axis=1, keepdims=True)
    d = (qn - 2.0 * jnp.dot(keys, q,
                            preferred_element_type=jnp.float32)) + kn
    g = d.shape[0] // 4
    a, b, c, e = d[:g], d[g:2 * g], d[2 * g:3 * g], d[3 * g:]
    lo1, hi1 = jnp.minimum(a, b), jnp.maximum(a, b)
    lo2, hi2 = jnp.minimum(c, e), jnp.maximum(c, e)
    v1 = jnp.minimum(lo1, lo2)
    v4 = jnp.maximum(hi1, hi2)
    u = jnp.maximum(lo1, lo2)
    z = jnp.minimum(hi1, hi2)
    v2 = jnp.minimum(u, z)
    v3 = jnp.maximum(u, z)
    work, s2, s3, s4 = v1, v2, v3, v4
    for i in range(_K):
        t = jnp.min(work, axis=0, keepdims=True)
        if i < _K - 1:
            m = work <= t
            work = jnp.where(m, s2, work)
            s2 = jnp.where(m, s3, s2)
            s3 = jnp.where(m, s4, s3)
            s4 = jnp.where(m, _BIG, s4)
    w = jnp.where(d <= t, jnp.exp(-_ALPHA * d), 0.0)
    nd = jnp.dot(lo_ref[...], w, precision=jax.lax.Precision.HIGHEST,
                 preferred_element_type=jnp.float32)
    o_ref[...] = nd[0] / (nd[1] + 1e-8)


def _im2col(x, ch):
    d, h, w = x.shape[1], x.shape[2], x.shape[3]
    xp = jnp.pad(x, ((0, 0), (1, 1), (1, 1), (1, 1)))
    cols = [xp[:, dz:dz + d, dy:dy + h, dx:dx + w]
            for dz in range(3) for dy in range(3) for dx in range(3)]
    return jnp.stack(cols).reshape(27 * ch, d * h * w)


def kernel(bg_prob, ed_prob, w1, b1, g1, be1, w2, b2, g2, be2, w3, b3,
           key_store, store_labels, context_mask, add_mode):
    B, _, D, H, W = bg_prob.shape
    N = B * D * H * W
    C = w3.shape[0]
    K = key_store.shape[0]

    x = jnp.concatenate([bg_prob, ed_prob], axis=1).reshape(2, D, H, W)
    x1 = _im2col(x, 2)
    x1 = jnp.pad(x1, ((0, 2), (0, 0)))
    w1m = jnp.transpose(w1, (2, 3, 4, 1, 0)).reshape(54, 16).T
    w1m = jnp.pad(w1m, ((0, 0), (0, 2)))

    w29 = jnp.transpose(w2, (3, 4, 0, 2, 1)).reshape(9 * 32, 48)
    w3m = w3.reshape(C, 32)

    lat = pl.pallas_call(
        _enc_body,
        out_shape=jax.ShapeDtypeStruct((C, N), jnp.float32),
    )(x1, w1m, b1.reshape(16, 1), g1.reshape(16, 1), be1.reshape(16, 1),
      w29, b2.reshape(32, 1), g2.reshape(32, 1), be2.reshape(32, 1),
      w3m, b3.reshape(C, 1))

    lbl_ones = jnp.zeros((8, K), jnp.float32)
    lbl_ones = lbl_ones.at[0].set(store_labels).at[1].set(1.0)

    R = 512
    prob = pl.pallas_call(
        _knn_body,
        grid=(N // R,),
        in_specs=[
            pl.BlockSpec((C, R), lambda i: (0, i)),
            pl.BlockSpec((K, C), lambda i: (0, 0)),
            pl.BlockSpec((8, K), lambda i: (0, 0)),
        ],
        out_specs=pl.BlockSpec((R,), lambda i: (i,)),
        out_shape=jax.ShapeDtypeStruct((N,), jnp.float32),
        compiler_params=pltpu.CompilerParams(
            dimension_semantics=("parallel",)),
    )(lat, key_store, lbl_ones)

    return prob.reshape(B, D, H, W)


# speedup vs baseline: 1.1034x; 1.1034x over previous
import jax
import jax.numpy as jnp
from jax.experimental import pallas as pl
from jax.experimental.pallas import tpu as pltpu

_ALPHA = 10.0
_K = 10
_BIG = 3.0e38


def _shift_cols(a, s, n):
    if s == 0:
        return a
    c = a.shape[0]
    if s > 0:
        return jnp.concatenate([a[:, s:], jnp.zeros((c, s), a.dtype)], axis=1)
    return jnp.concatenate([jnp.zeros((c, -s), a.dtype), a[:, :s]], axis=1)


def _enc_body(x1_ref, w1_ref, b1_ref, g1_ref, be1_ref,
              w29_ref, b2_ref, g2_ref, be2_ref, w3_ref, b3_ref, o_ref):
    n = x1_ref.shape[1]
    h = jnp.dot(w1_ref[...], x1_ref[...],
                preferred_element_type=jnp.float32)
    h = h + b1_ref[...]
    m = jnp.mean(h, axis=1, keepdims=True)
    v = jnp.mean((h - m) ** 2, axis=1, keepdims=True)
    h = (h - m) / jnp.sqrt(v + 1e-5) * g1_ref[...] + be1_ref[...]
    h = jnp.maximum(h, 0.0)

    col = jax.lax.broadcasted_iota(jnp.int32, (1, n), 1)
    xc = col % 32
    yc = (col // 32) % 32

    acc = jnp.zeros((32, n), jnp.float32)
    j = 0
    for ey in (-1, 0, 1):
        my = ((yc + ey) >= 0) & ((yc + ey) < 32)
        for ex in (-1, 0, 1):
            mask = (my & ((xc + ex) >= 0) & ((xc + ex) < 32)).astype(h.dtype)
            sxy = _shift_cols(h, 32 * ey + ex, n) * mask
            stk = jnp.concatenate(
                [_shift_cols(sxy, 1024 * ez, n) for ez in (-1, 0, 1)], axis=0)
            acc = acc + jnp.dot(w29_ref[32 * j:32 * (j + 1), :], stk,
                                preferred_element_type=jnp.float32)
            j += 1

    h2 = acc + b2_ref[...]
    m = jnp.mean(h2, axis=1, keepdims=True)
    v = jnp.mean((h2 - m) ** 2, axis=1, keepdims=True)
    h2 = (h2 - m) / jnp.sqrt(v + 1e-5) * g2_ref[...] + be2_ref[...]
    h2 = jnp.maximum(h2, 0.0)

    lat = jnp.dot(w3_ref[...], h2,
                  preferred_element_type=jnp.float32)
    lat = lat + b3_ref[...]
    norm = jnp.sqrt(jnp.sum(lat * lat, axis=0, keepdims=True))
    o_ref[...] = lat / jnp.maximum(norm, 1e-12)


def _knn_body(q_ref, k_ref, lo_ref, o_ref):
    q = q_ref[...]
    keys = k_ref[...]
    qn = jnp.sum(q * q, axis=0, keepdims=True)
    kn = jnp.sum(keys * keys, axis=1, keepdims=True)
    d = (qn - 2.0 * jnp.dot(keys, q,
                            preferred_element_type=jnp.float32)) + kn
    work = d
    for i in range(_K):
        t = jnp.min(work, axis=0, keepdims=True)
        if i < _K - 1:
            work = jnp.where(work <= t, _BIG, work)
    w = jnp.where(d <= t, jnp.exp(-_ALPHA * d), 0.0)
    nd = jnp.dot(lo_ref[...], w, precision=jax.lax.Precision.HIGHEST,
                 preferred_element_type=jnp.float32)
    o_ref[...] = nd[0] / (nd[1] + 1e-8)


def _im2col(x, ch):
    d, h, w = x.shape[1], x.shape[2], x.shape[3]
    xp = jnp.pad(x, ((0, 0), (1, 1), (1, 1), (1, 1)))
    cols = [xp[:, dz:dz + d, dy:dy + h, dx:dx + w]
            for dz in range(3) for dy in range(3) for dx in range(3)]
    return jnp.stack(cols).reshape(27 * ch, d * h * w)


def kernel(bg_prob, ed_prob, w1, b1, g1, be1, w2, b2, g2, be2, w3, b3,
           key_store, store_labels, context_mask, add_mode):
    B, _, D, H, W = bg_prob.shape
    N = B * D * H * W
    C = w3.shape[0]
    K = key_store.shape[0]

    x = jnp.concatenate([bg_prob, ed_prob], axis=1).reshape(2, D, H, W)
    x1 = _im2col(x, 2)
    x1 = jnp.pad(x1, ((0, 2), (0, 0)))
    w1m = jnp.transpose(w1, (2, 3, 4, 1, 0)).reshape(54, 16).T
    w1m = jnp.pad(w1m, ((0, 0), (0, 2)))

    w29 = jnp.transpose(w2, (3, 4, 0, 2, 1)).reshape(9 * 32, 48)
    w3m = w3.reshape(C, 32)

    lat = pl.pallas_call(
        _enc_body,
        out_shape=jax.ShapeDtypeStruct((C, N), jnp.float32),
    )(x1, w1m, b1.reshape(16, 1), g1.reshape(16, 1), be1.reshape(16, 1),
      w29, b2.reshape(32, 1), g2.reshape(32, 1), be2.reshape(32, 1),
      w3m, b3.reshape(C, 1))

    lbl_ones = jnp.zeros((8, K), jnp.float32)
    lbl_ones = lbl_ones.at[0].set(store_labels).at[1].set(1.0)

    R = 512
    prob = pl.pallas_call(
        _knn_body,
        grid=(N // R,),
        in_specs=[
            pl.BlockSpec((C, R), lambda i: (0, i)),
            pl.BlockSpec((K, C), lambda i: (0, 0)),
            pl.BlockSpec((8, K), lambda i: (0, 0)),
        ],
        out_specs=pl.BlockSpec((R,), lambda i: (i,)),
        out_shape=jax.ShapeDtypeStruct((N,), jnp.float32),
        compiler_params=pltpu.CompilerParams(
            dimension_semantics=("parallel",)),
    )(lat, key_store, lbl_ones)

    return prob.reshape(B, D, H, W)


# speedup vs baseline: 1.2440x; 1.1275x over previous
import jax
import jax.numpy as jnp
from jax.experimental import pallas as pl
from jax.experimental.pallas import tpu as pltpu

_ALPHA = 10.0
_K = 10
_BIG = 3.0e38


def _shift_cols(a, s, n):
    if s == 0:
        return a
    c = a.shape[0]
    if s > 0:
        return jnp.concatenate([a[:, s:], jnp.zeros((c, s), a.dtype)], axis=1)
    return jnp.concatenate([jnp.zeros((c, -s), a.dtype), a[:, :s]], axis=1)


def _enc_body(x1_ref, w1_ref, b1_ref, g1_ref, be1_ref,
              w29_ref, b2_ref, g2_ref, be2_ref, w3_ref, b3_ref, o_ref):
    n = x1_ref.shape[1]
    h = jnp.dot(w1_ref[...], x1_ref[...],
                preferred_element_type=jnp.float32)
    h = h + b1_ref[...]
    m = jnp.mean(h, axis=1, keepdims=True)
    v = jnp.mean((h - m) ** 2, axis=1, keepdims=True)
    h = (h - m) / jnp.sqrt(v + 1e-5) * g1_ref[...] + be1_ref[...]
    h = jnp.maximum(h, 0.0)

    col = jax.lax.broadcasted_iota(jnp.int32, (1, n), 1)
    xc = col % 32
    yc = (col // 32) % 32

    acc = jnp.zeros((32, n), jnp.float32)
    j = 0
    for ey in (-1, 0, 1):
        my = ((yc + ey) >= 0) & ((yc + ey) < 32)
        for ex in (-1, 0, 1):
            mask = (my & ((xc + ex) >= 0) & ((xc + ex) < 32)).astype(h.dtype)
            sxy = _shift_cols(h, 32 * ey + ex, n) * mask
            stk = jnp.concatenate(
                [_shift_cols(sxy, 1024 * ez, n) for ez in (-1, 0, 1)], axis=0)
            acc = acc + jnp.dot(w29_ref[32 * j:32 * (j + 1), :], stk,
                                preferred_element_type=jnp.float32)
            j += 1

    h2 = acc + b2_ref[...]
    m = jnp.mean(h2, axis=1, keepdims=True)
    v = jnp.mean((h2 - m) ** 2, axis=1, keepdims=True)
    h2 = (h2 - m) / jnp.sqrt(v + 1e-5) * g2_ref[...] + be2_ref[...]
    h2 = jnp.maximum(h2, 0.0)

    lat = jnp.dot(w3_ref[...], h2,
                  preferred_element_type=jnp.float32)
    lat = lat + b3_ref[...]
    norm = jnp.sqrt(jnp.sum(lat * lat, axis=0, keepdims=True))
    o_ref[...] = lat / jnp.maximum(norm, 1e-12)


def _knn_body(q_ref, k_ref, lo_ref, o_ref):
    q = q_ref[...]
    keys = k_ref[...]
    qn = jnp.sum(q * q, axis=0, keepdims=True)
    kn = jnp.sum(keys * keys, axis=1, keepdims=True)
    d = (qn - 2.0 * jnp.dot(keys, q,
                            preferred_element_type=jnp.float32)) + kn
    t = jnp.min(d, axis=0, keepdims=True)
    for _ in range(_K - 1):
        t = jnp.min(jnp.where(d <= t, _BIG, d), axis=0, keepdims=True)
    w = jnp.where(d <= t, jnp.exp(-_ALPHA * d), 0.0)
    num = jnp.sum(w * lo_ref[...], axis=0)
    den = jnp.sum(w, axis=0)
    o_ref[...] = num / (den + 1e-8)


def _im2col(x, ch):
    d, h, w = x.shape[1], x.shape[2], x.shape[3]
    xp = jnp.pad(x, ((0, 0), (1, 1), (1, 1), (1, 1)))
    cols = [xp[:, dz:dz + d, dy:dy + h, dx:dx + w]
            for dz in range(3) for dy in range(3) for dx in range(3)]
    return jnp.stack(cols).reshape(27 * ch, d * h * w)


def kernel(bg_prob, ed_prob, w1, b1, g1, be1, w2, b2, g2, be2, w3, b3,
           key_store, store_labels, context_mask, add_mode):
    B, _, D, H, W = bg_prob.shape
    N = B * D * H * W
    C = w3.shape[0]
    K = key_store.shape[0]

    x = jnp.concatenate([bg_prob, ed_prob], axis=1).reshape(2, D, H, W)
    x1 = _im2col(x, 2)
    x1 = jnp.pad(x1, ((0, 2), (0, 0)))
    w1m = jnp.transpose(w1, (2, 3, 4, 1, 0)).reshape(54, 16).T
    w1m = jnp.pad(w1m, ((0, 0), (0, 2)))

    w29 = jnp.transpose(w2, (3, 4, 0, 2, 1)).reshape(9 * 32, 48)
    w3m = w3.reshape(C, 32)

    lat = pl.pallas_call(
        _enc_body,
        out_shape=jax.ShapeDtypeStruct((C, N), jnp.float32),
    )(x1, w1m, b1.reshape(16, 1), g1.reshape(16, 1), be1.reshape(16, 1),
      w29, b2.reshape(32, 1), g2.reshape(32, 1), be2.reshape(32, 1),
      w3m, b3.reshape(C, 1))

    R = 512
    prob = pl.pallas_call(
        _knn_body,
        grid=(N // R,),
        in_specs=[
            pl.BlockSpec((C, R), lambda i: (0, i)),
            pl.BlockSpec((K, C), lambda i: (0, 0)),
            pl.BlockSpec((K, 1), lambda i: (0, 0)),
        ],
        out_specs=pl.BlockSpec((R,), lambda i: (i,)),
        out_shape=jax.ShapeDtypeStruct((N,), jnp.float32),
        compiler_params=pltpu.CompilerParams(
            dimension_semantics=("parallel",)),
    )(lat, key_store, store_labels.reshape(K, 1))

    return prob.reshape(B, D, H, W)


# speedup vs baseline: 1.3398x; 1.0770x over previous
import jax
import jax.numpy as jnp
from jax.experimental import pallas as pl
from jax.experimental.pallas import tpu as pltpu

_ALPHA = 10.0
_K = 10
_BIG = 3.0e38


def _shift_cols(a, s, n):
    if s == 0:
        return a
    c = a.shape[0]
    if s > 0:
        return jnp.concatenate([a[:, s:], jnp.zeros((c, s), a.dtype)], axis=1)
    return jnp.concatenate([jnp.zeros((c, -s), a.dtype), a[:, :s]], axis=1)


def _conv_shift(h, w_ref, co, masks, n):
    ci = h.shape[0]
    acc = jnp.zeros((co, n), jnp.float32)
    for j, (ey, ex) in enumerate((ey, ex) for ey in (-1, 0, 1)
                                 for ex in (-1, 0, 1)):
        sxy = _shift_cols(h, 32 * ey + ex, n) * masks[j]
        stk = jnp.concatenate(
            [_shift_cols(sxy, 1024 * ez, n) for ez in (-1, 0, 1)], axis=0)
        acc = acc + jnp.dot(w_ref[co * j:co * (j + 1), :], stk,
                            preferred_element_type=jnp.float32)
    return acc


def _enc_body(x_ref, w19_ref, b1_ref, g1_ref, be1_ref,
              w29_ref, b2_ref, g2_ref, be2_ref, w3_ref, b3_ref, o_ref):
    n = x_ref.shape[1]
    col = jax.lax.broadcasted_iota(jnp.int32, (1, n), 1)
    xc = col % 32
    yc = (col // 32) % 32
    masks = []
    for ey in (-1, 0, 1):
        my = ((yc + ey) >= 0) & ((yc + ey) < 32)
        for ex in (-1, 0, 1):
            masks.append(
                (my & ((xc + ex) >= 0) & ((xc + ex) < 32)).astype(jnp.float32))

    h = _conv_shift(x_ref[...], w19_ref, 16, masks, n)
    h = h + b1_ref[...]
    m = jnp.mean(h, axis=1, keepdims=True)
    v = jnp.mean((h - m) ** 2, axis=1, keepdims=True)
    h = (h - m) / jnp.sqrt(v + 1e-5) * g1_ref[...] + be1_ref[...]
    h = jnp.maximum(h, 0.0)

    h2 = _conv_shift(h, w29_ref, 32, masks, n) + b2_ref[...]
    m = jnp.mean(h2, axis=1, keepdims=True)
    v = jnp.mean((h2 - m) ** 2, axis=1, keepdims=True)
    h2 = (h2 - m) / jnp.sqrt(v + 1e-5) * g2_ref[...] + be2_ref[...]
    h2 = jnp.maximum(h2, 0.0)

    lat = jnp.dot(w3_ref[...], h2,
                  preferred_element_type=jnp.float32)
    lat = lat + b3_ref[...]
    norm = jnp.sqrt(jnp.sum(lat * lat, axis=0, keepdims=True))
    o_ref[...] = lat / jnp.maximum(norm, 1e-12)


def _knn_body(q_ref, k_ref, lo_ref, o_ref):
    q = q_ref[...]
    keys = k_ref[...]
    qn = jnp.sum(q * q, axis=0, keepdims=True)
    kn = jnp.sum(keys * keys, axis=1, keepdims=True)
    d = (qn - 2.0 * jnp.dot(keys, q,
                            preferred_element_type=jnp.float32)) + kn
    t = jnp.min(d, axis=0, keepdims=True)
    for _ in range(_K - 1):
        t = jnp.min(jnp.where(d <= t, _BIG, d), axis=0, keepdims=True)
    w = jnp.where(d <= t, jnp.exp(-_ALPHA * d), 0.0)
    num = jnp.sum(w * lo_ref[...], axis=0)
    den = jnp.sum(w, axis=0)
    o_ref[...] = num / (den + 1e-8)


def kernel(bg_prob, ed_prob, w1, b1, g1, be1, w2, b2, g2, be2, w3, b3,
           key_store, store_labels, context_mask, add_mode):
    B, _, D, H, W = bg_prob.shape
    N = B * D * H * W
    C = w3.shape[0]
    K = key_store.shape[0]

    x = jnp.concatenate([bg_prob, ed_prob], axis=1).reshape(2, N)
    x8 = jnp.pad(x, ((0, 6), (0, 0)))

    w19 = jnp.pad(jnp.transpose(w1, (3, 4, 0, 2, 1)),
                  ((0, 0), (0, 0), (0, 0), (0, 0), (0, 6))).reshape(9 * 16, 24)
    w29 = jnp.transpose(w2, (3, 4, 0, 2, 1)).reshape(9 * 32, 48)
    w3m = w3.reshape(C, 32)

    lat = pl.pallas_call(
        _enc_body,
        out_shape=jax.ShapeDtypeStruct((C, N), jnp.float32),
    )(x8, w19, b1.reshape(16, 1), g1.reshape(16, 1), be1.reshape(16, 1),
      w29, b2.reshape(32, 1), g2.reshape(32, 1), be2.reshape(32, 1),
      w3m, b3.reshape(C, 1))

    R = 512
    prob = pl.pallas_call(
        _knn_body,
        grid=(N // R,),
        in_specs=[
            pl.BlockSpec((C, R), lambda i: (0, i)),
            pl.BlockSpec((K, C), lambda i: (0, 0)),
            pl.BlockSpec((K, 1), lambda i: (0, 0)),
        ],
        out_specs=pl.BlockSpec((R,), lambda i: (i,)),
        out_shape=jax.ShapeDtypeStruct((N,), jnp.float32),
        compiler_params=pltpu.CompilerParams(
            dimension_semantics=("parallel",)),
    )(lat, key_store, store_labels.reshape(K, 1))

    return prob.reshape(B, D, H, W)


# speedup vs baseline: 1.3431x; 1.0025x over previous
import jax
import jax.numpy as jnp
from jax.experimental import pallas as pl
from jax.experimental.pallas import tpu as pltpu

_ALPHA = 10.0
_K = 10
_BIG = 3.0e38


def _shift_cols(a, s, n):
    if s == 0:
        return a
    c = a.shape[0]
    if s > 0:
        return jnp.concatenate([a[:, s:], jnp.zeros((c, s), a.dtype)], axis=1)
    return jnp.concatenate([jnp.zeros((c, -s), a.dtype), a[:, :s]], axis=1)


def _conv_shift(h, w_ref, co, masks, n):
    ci = h.shape[0]
    acc = jnp.zeros((co, n), jnp.float32)
    for j, (ey, ex) in enumerate((ey, ex) for ey in (-1, 0, 1)
                                 for ex in (-1, 0, 1)):
        sxy = _shift_cols(h, 32 * ey + ex, n) * masks[j]
        stk = jnp.concatenate(
            [_shift_cols(sxy, 1024 * ez, n) for ez in (-1, 0, 1)], axis=0)
        acc = acc + jnp.dot(w_ref[co * j:co * (j + 1), :], stk,
                            preferred_element_type=jnp.float32)
    return acc


def _mono_body(x_ref, w19_ref, b1_ref, g1_ref, be1_ref,
               w29_ref, b2_ref, g2_ref, be2_ref, w3_ref, b3_ref,
               k_ref, l_ref, o_ref, lat_ref):
    i = pl.program_id(0)
    r = o_ref.shape[0]

    @pl.when(i == 0)
    def _encode():
        _enc_into(x_ref, w19_ref, b1_ref, g1_ref, be1_ref,
                  w29_ref, b2_ref, g2_ref, be2_ref, w3_ref, b3_ref, lat_ref)

    q = lat_ref[:, pl.ds(i * r, r)]
    keys = k_ref[...]
    qn = jnp.sum(q * q, axis=0, keepdims=True)
    kn = jnp.sum(keys * keys, axis=1, keepdims=True)
    d = (qn - 2.0 * jnp.dot(keys, q,
                            preferred_element_type=jnp.float32)) + kn
    t = jnp.min(d, axis=0, keepdims=True)
    for _ in range(_K - 1):
        t = jnp.min(jnp.where(d <= t, _BIG, d), axis=0, keepdims=True)
    w = jnp.where(d <= t, jnp.exp(-_ALPHA * d), 0.0)
    num = jnp.sum(w * l_ref[...], axis=0)
    den = jnp.sum(w, axis=0)
    o_ref[...] = num / (den + 1e-8)


def _enc_into(x_ref, w19_ref, b1_ref, g1_ref, be1_ref,
              w29_ref, b2_ref, g2_ref, be2_ref, w3_ref, b3_ref, o_ref):
    n = x_ref.shape[1]
    col = jax.lax.broadcasted_iota(jnp.int32, (1, n), 1)
    xc = col % 32
    yc = (col // 32) % 32
    masks = []
    for ey in (-1, 0, 1):
        my = ((yc + ey) >= 0) & ((yc + ey) < 32)
        for ex in (-1, 0, 1):
            masks.append(
                (my & ((xc + ex) >= 0) & ((xc + ex) < 32)).astype(jnp.float32))

    h = _conv_shift(x_ref[...], w19_ref, 16, masks, n)
    h = h + b1_ref[...]
    m = jnp.mean(h, axis=1, keepdims=True)
    v = jnp.mean((h - m) ** 2, axis=1, keepdims=True)
    h = (h - m) / jnp.sqrt(v + 1e-5) * g1_ref[...] + be1_ref[...]
    h = jnp.maximum(h, 0.0)

    h2 = _conv_shift(h, w29_ref, 32, masks, n) + b2_ref[...]
    m = jnp.mean(h2, axis=1, keepdims=True)
    v = jnp.mean((h2 - m) ** 2, axis=1, keepdims=True)
    h2 = (h2 - m) / jnp.sqrt(v + 1e-5) * g2_ref[...] + be2_ref[...]
    h2 = jnp.maximum(h2, 0.0)

    lat = jnp.dot(w3_ref[...], h2,
                  preferred_element_type=jnp.float32)
    lat = lat + b3_ref[...]
    norm = jnp.sqrt(jnp.sum(lat * lat, axis=0, keepdims=True))
    o_ref[...] = lat / jnp.maximum(norm, 1e-12)


def kernel(bg_prob, ed_prob, w1, b1, g1, be1, w2, b2, g2, be2, w3, b3,
           key_store, store_labels, context_mask, add_mode):
    B, _, D, H, W = bg_prob.shape
    N = B * D * H * W
    C = w3.shape[0]
    K = key_store.shape[0]

    x = jnp.concatenate([bg_prob, ed_prob], axis=1).reshape(2, N)
    x8 = jnp.pad(x, ((0, 6), (0, 0)))

    w19 = jnp.pad(jnp.transpose(w1, (3, 4, 0, 2, 1)),
                  ((0, 0), (0, 0), (0, 0), (0, 0), (0, 6))).reshape(9 * 16, 24)
    w29 = jnp.transpose(w2, (3, 4, 0, 2, 1)).reshape(9 * 32, 48)
    w3m = w3.reshape(C, 32)

    R = 512
    full = lambda i: (0, 0)
    prob = pl.pallas_call(
        _mono_body,
        grid=(N // R,),
        in_specs=[
            pl.BlockSpec((8, N), full),
            pl.BlockSpec((9 * 16, 24), full),
            pl.BlockSpec((16, 1), full),
            pl.BlockSpec((16, 1), full),
            pl.BlockSpec((16, 1), full),
            pl.BlockSpec((9 * 32, 48), full),
            pl.BlockSpec((32, 1), full),
            pl.BlockSpec((32, 1), full),
            pl.BlockSpec((32, 1), full),
            pl.BlockSpec((C, 32), full),
            pl.BlockSpec((C, 1), full),
            pl.BlockSpec((K, C), full),
            pl.BlockSpec((K, 1), full),
        ],
        out_specs=pl.BlockSpec((R,), lambda i: (i,)),
        out_shape=jax.ShapeDtypeStruct((N,), jnp.float32),
        scratch_shapes=[pltpu.VMEM((C, N), jnp.float32)],
    )(x8, w19, b1.reshape(16, 1), g1.reshape(16, 1), be1.reshape(16, 1),
      w29, b2.reshape(32, 1), g2.reshape(32, 1), be2.reshape(32, 1),
      w3m, b3.reshape(C, 1), key_store, store_labels.reshape(K, 1))

    return prob.reshape(B, D, H, W)


# speedup vs baseline: 1.3624x; 1.0143x over previous
import jax
import jax.numpy as jnp
from jax.experimental import pallas as pl
from jax.experimental.pallas import tpu as pltpu

_ALPHA = 10.0
_K = 10
_BIG = 3.0e38


def _shift_cols(a, s, n):
    if s == 0:
        return a
    c = a.shape[0]
    if s > 0:
        return jnp.concatenate([a[:, s:], jnp.zeros((c, s), a.dtype)], axis=1)
    return jnp.concatenate([jnp.zeros((c, -s), a.dtype), a[:, :s]], axis=1)


def _conv_shift(h, w_ref, co, masks, n):
    ci = h.shape[0]
    acc = jnp.zeros((co, n), jnp.float32)
    for j, (ey, ex) in enumerate((ey, ex) for ey in (-1, 0, 1)
                                 for ex in (-1, 0, 1)):
        sxy = _shift_cols(h, 32 * ey + ex, n) * masks[j]
        stk = jnp.concatenate(
            [_shift_cols(sxy, 1024 * ez, n) for ez in (-1, 0, 1)], axis=0)
        acc = acc + jnp.dot(w_ref[co * j:co * (j + 1), :], stk,
                            preferred_element_type=jnp.float32)
    return acc


def _mono_body(x_ref, w19_ref, b1_ref, g1_ref, be1_ref,
               w29_ref, b2_ref, g2_ref, be2_ref, w3_ref, b3_ref,
               k_ref, l_ref, o_ref, lat_ref):
    i = pl.program_id(0)
    r = o_ref.shape[0]

    @pl.when(i == 0)
    def _encode():
        _enc_into(x_ref, w19_ref, b1_ref, g1_ref, be1_ref,
                  w29_ref, b2_ref, g2_ref, be2_ref, w3_ref, b3_ref, lat_ref)

    q = lat_ref[:, pl.ds(i * r, r)]
    keys = k_ref[...]
    qn = jnp.sum(q * q, axis=0, keepdims=True)
    kn = jnp.sum(keys * keys, axis=1, keepdims=True)
    d = (qn - 2.0 * jnp.dot(keys, q,
                            preferred_element_type=jnp.float32)) + kn
    g = d.shape[0] // 2
    p = jnp.minimum(d[:g], d[g:])
    s = jnp.maximum(d[:g], d[g:])
    t = jnp.min(p, axis=0, keepdims=True)
    for _ in range(_K - 1):
        m = p <= t
        p = jnp.where(m, s, p)
        s = jnp.where(m, _BIG, s)
        t = jnp.min(p, axis=0, keepdims=True)
    w = jnp.where(d <= t, jnp.exp(-_ALPHA * d), 0.0)
    num = jnp.sum(w * l_ref[...], axis=0)
    den = jnp.sum(w, axis=0)
    o_ref[...] = num / (den + 1e-8)


def _enc_into(x_ref, w19_ref, b1_ref, g1_ref, be1_ref,
              w29_ref, b2_ref, g2_ref, be2_ref, w3_ref, b3_ref, o_ref):
    n = x_ref.shape[1]
    col = jax.lax.broadcasted_iota(jnp.int32, (1, n), 1)
    xc = col % 32
    yc = (col // 32) % 32
    masks = []
    for ey in (-1, 0, 1):
        my = ((yc + ey) >= 0) & ((yc + ey) < 32)
        for ex in (-1, 0, 1):
            masks.append(
                (my & ((xc + ex) >= 0) & ((xc + ex) < 32)).astype(jnp.float32))

    h = _conv_shift(x_ref[...], w19_ref, 16, masks, n)
    h = h + b1_ref[...]
    m = jnp.mean(h, axis=1, keepdims=True)
    v = jnp.mean((h - m) ** 2, axis=1, keepdims=True)
    h = (h - m) / jnp.sqrt(v + 1e-5) * g1_ref[...] + be1_ref[...]
    h = jnp.maximum(h, 0.0)

    h2 = _conv_shift(h, w29_ref, 32, masks, n) + b2_ref[...]
    m = jnp.mean(h2, axis=1, keepdims=True)
    v = jnp.mean((h2 - m) ** 2, axis=1, keepdims=True)
    h2 = (h2 - m) / jnp.sqrt(v + 1e-5) * g2_ref[...] + be2_ref[...]
    h2 = jnp.maximum(h2, 0.0)

    lat = jnp.dot(w3_ref[...], h2,
                  preferred_element_type=jnp.float32)
    lat = lat + b3_ref[...]
    norm = jnp.sqrt(jnp.sum(lat * lat, axis=0, keepdims=True))
    o_ref[...] = lat / jnp.maximum(norm, 1e-12)


def kernel(bg_prob, ed_prob, w1, b1, g1, be1, w2, b2, g2, be2, w3, b3,
           key_store, store_labels, context_mask, add_mode):
    B, _, D, H, W = bg_prob.shape
    N = B * D * H * W
    C = w3.shape[0]
    K = key_store.shape[0]

    x = jnp.concatenate([bg_prob, ed_prob], axis=1).reshape(2, N)
    x8 = jnp.pad(x, ((0, 6), (0, 0)))

    w19 = jnp.pad(jnp.transpose(w1, (3, 4, 0, 2, 1)),
                  ((0, 0), (0, 0), (0, 0), (0, 0), (0, 6))).reshape(9 * 16, 24)
    w29 = jnp.transpose(w2, (3, 4, 0, 2, 1)).reshape(9 * 32, 48)
    w3m = w3.reshape(C, 32)

    R = 512
    full = lambda i: (0, 0)
    prob = pl.pallas_call(
        _mono_body,
        grid=(N // R,),
        in_specs=[
            pl.BlockSpec((8, N), full),
            pl.BlockSpec((9 * 16, 24), full),
            pl.BlockSpec((16, 1), full),
            pl.BlockSpec((16, 1), full),
            pl.BlockSpec((16, 1), full),
            pl.BlockSpec((9 * 32, 48), full),
            pl.BlockSpec((32, 1), full),
            pl.BlockSpec((32, 1), full),
            pl.BlockSpec((32, 1), full),
            pl.BlockSpec((C, 32), full),
            pl.BlockSpec((C, 1), full),
            pl.BlockSpec((K, C), full),
            pl.BlockSpec((K, 1), full),
        ],
        out_specs=pl.BlockSpec((R,), lambda i: (i,)),
        out_shape=jax.ShapeDtypeStruct((N,), jnp.float32),
        scratch_shapes=[pltpu.VMEM((C, N), jnp.float32)],
    )(x8, w19, b1.reshape(16, 1), g1.reshape(16, 1), be1.reshape(16, 1),
      w29, b2.reshape(32, 1), g2.reshape(32, 1), be2.reshape(32, 1),
      w3m, b3.reshape(C, 1), key_store, store_labels.reshape(K, 1))

    return prob.reshape(B, D, H, W)


# speedup vs baseline: 1.3907x; 1.0208x over previous
import jax
import jax.numpy as jnp
from jax.experimental import pallas as pl
from jax.experimental.pallas import tpu as pltpu

_ALPHA = 10.0
_K = 10
_BIG = 3.0e38


def _shift_cols(a, s, n):
    if s == 0:
        return a
    c = a.shape[0]
    if s > 0:
        return jnp.concatenate([a[:, s:], jnp.zeros((c, s), a.dtype)], axis=1)
    return jnp.concatenate([jnp.zeros((c, -s), a.dtype), a[:, :s]], axis=1)


def _conv_shift(h, w_ref, co, masks, n):
    ci = h.shape[0]
    acc = jnp.zeros((co, n), jnp.float32)
    for j, (ey, ex) in enumerate((ey, ex) for ey in (-1, 0, 1)
                                 for ex in (-1, 0, 1)):
        sxy = _shift_cols(h, 32 * ey + ex, n) * masks[j]
        stk = jnp.concatenate(
            [_shift_cols(sxy, 1024 * ez, n) for ez in (-1, 0, 1)], axis=0)
        acc = acc + jnp.dot(w_ref[co * j:co * (j + 1), :], stk,
                            preferred_element_type=jnp.float32)
    return acc


def _mono_body(x_ref, w19_ref, b1_ref, g1_ref, be1_ref,
               w29_ref, b2_ref, g2_ref, be2_ref, w3_ref, b3_ref,
               k_ref, l_ref, o_ref, lat_ref):
    i = pl.program_id(0)
    r = o_ref.shape[0]

    @pl.when(i == 0)
    def _encode():
        _enc_into(x_ref, w19_ref, b1_ref, g1_ref, be1_ref,
                  w29_ref, b2_ref, g2_ref, be2_ref, w3_ref, b3_ref, lat_ref)

    q = lat_ref[:, pl.ds(i * r, r)]
    keys = k_ref[...]
    qn = jnp.sum(q * q, axis=0, keepdims=True)
    kn = jnp.sum(keys * keys, axis=1, keepdims=True)
    d = (qn - 2.0 * jnp.dot(keys, q,
                            preferred_element_type=jnp.float32)) + kn
    g = d.shape[0] // 2
    p = jnp.minimum(d[:g], d[g:])
    s = jnp.maximum(d[:g], d[g:])
    t = jnp.min(p, axis=0, keepdims=True)
    for _ in range(_K - 1):
        m = p <= t
        p = jnp.where(m, s, p)
        s = jnp.where(m, _BIG, s)
        t = jnp.min(p, axis=0, keepdims=True)
    w = jnp.where(d <= t, jnp.exp(-_ALPHA * d), 0.0)
    num = jnp.sum(w * l_ref[...], axis=0)
    den = jnp.sum(w, axis=0)
    o_ref[...] = num / (den + 1e-8)


def _enc_into(x_ref, w19_ref, b1_ref, g1_ref, be1_ref,
              w29_ref, b2_ref, g2_ref, be2_ref, w3_ref, b3_ref, o_ref):
    n = x_ref.shape[1]
    col = jax.lax.broadcasted_iota(jnp.int32, (1, n), 1)
    xc = col % 32
    yc = (col // 32) % 32
    masks = []
    for ey in (-1, 0, 1):
        my = ((yc + ey) >= 0) & ((yc + ey) < 32)
        for ex in (-1, 0, 1):
            masks.append(
                (my & ((xc + ex) >= 0) & ((xc + ex) < 32)).astype(jnp.float32))

    h = _conv_shift(x_ref[...], w19_ref, 16, masks, n)
    h = h + b1_ref[...]
    m = jnp.mean(h, axis=1, keepdims=True)
    v = jnp.mean((h - m) ** 2, axis=1, keepdims=True)
    h = (h - m) / jnp.sqrt(v + 1e-5) * g1_ref[...] + be1_ref[...]
    h = jnp.maximum(h, 0.0)

    h2 = _conv_shift(h, w29_ref, 32, masks, n) + b2_ref[...]
    m = jnp.mean(h2, axis=1, keepdims=True)
    v = jnp.mean((h2 - m) ** 2, axis=1, keepdims=True)
    h2 = (h2 - m) / jnp.sqrt(v + 1e-5) * g2_ref[...] + be2_ref[...]
    h2 = jnp.maximum(h2, 0.0)

    lat = jnp.dot(w3_ref[...], h2,
                  preferred_element_type=jnp.float32)
    lat = lat + b3_ref[...]
    norm = jnp.sqrt(jnp.sum(lat * lat, axis=0, keepdims=True))
    o_ref[...] = lat / jnp.maximum(norm, 1e-12)


def kernel(bg_prob, ed_prob, w1, b1, g1, be1, w2, b2, g2, be2, w3, b3,
           key_store, store_labels, context_mask, add_mode):
    B, _, D, H, W = bg_prob.shape
    N = B * D * H * W
    C = w3.shape[0]
    K = key_store.shape[0]

    x = jnp.concatenate([bg_prob, ed_prob], axis=1).reshape(2, N)
    x8 = jnp.pad(x, ((0, 6), (0, 0)))

    w19 = jnp.pad(jnp.transpose(w1, (3, 4, 0, 2, 1)),
                  ((0, 0), (0, 0), (0, 0), (0, 0), (0, 6))).reshape(9 * 16, 24)
    w29 = jnp.transpose(w2, (3, 4, 0, 2, 1)).reshape(9 * 32, 48)
    w3m = w3.reshape(C, 32)

    R = 1024
    full = lambda i: (0, 0)
    prob = pl.pallas_call(
        _mono_body,
        grid=(N // R,),
        in_specs=[
            pl.BlockSpec((8, N), full),
            pl.BlockSpec((9 * 16, 24), full),
            pl.BlockSpec((16, 1), full),
            pl.BlockSpec((16, 1), full),
            pl.BlockSpec((16, 1), full),
            pl.BlockSpec((9 * 32, 48), full),
            pl.BlockSpec((32, 1), full),
            pl.BlockSpec((32, 1), full),
            pl.BlockSpec((32, 1), full),
            pl.BlockSpec((C, 32), full),
            pl.BlockSpec((C, 1), full),
            pl.BlockSpec((K, C), full),
            pl.BlockSpec((K, 1), full),
        ],
        out_specs=pl.BlockSpec((R,), lambda i: (i,)),
        out_shape=jax.ShapeDtypeStruct((N,), jnp.float32),
        scratch_shapes=[pltpu.VMEM((C, N), jnp.float32)],
    )(x8, w19, b1.reshape(16, 1), g1.reshape(16, 1), be1.reshape(16, 1),
      w29, b2.reshape(32, 1), g2.reshape(32, 1), be2.reshape(32, 1),
      w3m, b3.reshape(C, 1), key_store, store_labels.reshape(K, 1))

    return prob.reshape(B, D, H, W)
